# baseline (device time: 57644 ns/iter reference)
import math

import jax
import jax.numpy as jnp
from jax import lax
from jax.experimental import pallas as pl
from jax.experimental.pallas import tpu as pltpu

N_DEV = 16

_RING = [0, 4, 8, 12, 13, 9, 5, 1, 2, 6, 10, 14, 15, 11, 7, 3]
_INV = [0] * N_DEV
for _pos, _dev in enumerate(_RING):
    _INV[_dev] = _pos
_NEXT = [_RING[(_INV[d] + 1) % N_DEV] for d in range(N_DEV)]
_PREV = [_RING[(_INV[d] - 1) % N_DEV] for d in range(N_DEV)]

N_RIGHT = 8
N_LEFT = 7


def _select(idx, table):
    out = jnp.int32(table[0])
    for p in range(1, N_DEV):
        out = jnp.where(idx == p, jnp.int32(table[p]), out)
    return out


def kernel(q, k, v):
    s, d = q.shape
    scale = 1.0 / math.sqrt(d)

    def body(q_ref, k_ref, v_ref, out_ref, kbuf, vbuf, send_sems, recv_sems):
        my = lax.axis_index("i")
        right = _select(my, _NEXT)
        left = _select(my, _PREV)

        barrier_sem = pltpu.get_barrier_semaphore()
        for nbr in (left, right):
            pl.semaphore_signal(
                barrier_sem, inc=1,
                device_id=(nbr,), device_id_type=pl.DeviceIdType.MESH,
            )
        pl.semaphore_wait(barrier_sem, 2)

        qb = q_ref[...].astype(jnp.bfloat16)
        kbuf[0, :, :] = k_ref[...].astype(jnp.bfloat16)
        vbuf[0, :, :] = v_ref[...].astype(jnp.bfloat16)

        m = jnp.full((s, 1), -jnp.inf, dtype=jnp.float32)
        l = jnp.zeros((s, 1), dtype=jnp.float32)
        acc = jnp.zeros((s, d), dtype=jnp.float32)

        def attend(slot, m, l, acc):
            kh = kbuf[slot, :, :]
            vh = vbuf[slot, :, :]
            scores = lax.dot_general(
                qb, kh, (((1,), (1,)), ((), ())),
                preferred_element_type=jnp.float32,
            ) * scale
            m_new = jnp.maximum(m, jnp.max(scores, axis=1, keepdims=True))
            p = jnp.exp(scores - m_new)
            corr = jnp.exp(m - m_new)
            l = l * corr + jnp.sum(p, axis=1, keepdims=True)
            acc = acc * corr + lax.dot_general(
                p.astype(jnp.bfloat16), vh, (((1,), (0,)), ((), ())),
                preferred_element_type=jnp.float32,
            )
            return m_new, l, acc

        def rdma(buf, src_slot, dst_slot, sem_idx, dev):
            return pltpu.make_async_remote_copy(
                src_ref=buf.at[src_slot],
                dst_ref=buf.at[dst_slot],
                send_sem=send_sems.at[sem_idx],
                recv_sem=recv_sems.at[sem_idx],
                device_id=(dev,),
                device_id_type=pl.DeviceIdType.MESH,
            )

        started = []
        prev = {}
        for t in range(N_RIGHT + 1):
            if t > 0:
                prev[("r", "k")].wait_recv()
            if t < N_RIGHT:
                r = rdma(kbuf, t, t + 1, 4 * t + 0, right)
                r.start()
                started.append(r)
                prev[("r", "k")] = r
            if t > 0:
                prev[("r", "v")].wait_recv()
            if t < N_RIGHT:
                r = rdma(vbuf, t, t + 1, 4 * t + 1, right)
                r.start()
                started.append(r)
                prev[("r", "v")] = r

            if 0 < t <= N_LEFT:
                prev[("l", "k")].wait_recv()
            if t < N_LEFT:
                src = 0 if t == 0 else 8 + t
                r = rdma(kbuf, src, 9 + t, 4 * t + 2, left)
                r.start()
                started.append(r)
                prev[("l", "k")] = r
            if 0 < t <= N_LEFT:
                prev[("l", "v")].wait_recv()
            if t < N_LEFT:
                src = 0 if t == 0 else 8 + t
                r = rdma(vbuf, src, 9 + t, 4 * t + 3, left)
                r.start()
                started.append(r)
                prev[("l", "v")] = r

            if t == 0:
                m, l, acc = attend(0, m, l, acc)
            else:
                m, l, acc = attend(t, m, l, acc)
                if t <= N_LEFT:
                    m, l, acc = attend(8 + t, m, l, acc)

        for r in started:
            r.wait_send()

        out_ref[...] = acc / l

    return pl.pallas_call(
        body,
        out_shape=jax.ShapeDtypeStruct((s, d), jnp.float32),
        in_specs=[
            pl.BlockSpec(memory_space=pltpu.VMEM),
            pl.BlockSpec(memory_space=pltpu.VMEM),
            pl.BlockSpec(memory_space=pltpu.VMEM),
        ],
        out_specs=pl.BlockSpec(memory_space=pltpu.VMEM),
        scratch_shapes=[
            pltpu.VMEM((N_DEV, s, d), jnp.bfloat16),
            pltpu.VMEM((N_DEV, s, d), jnp.bfloat16),
            pltpu.SemaphoreType.DMA((4 * N_RIGHT,)),
            pltpu.SemaphoreType.DMA((4 * N_RIGHT,)),
        ],
        compiler_params=pltpu.CompilerParams(collective_id=0),
    )(q, k, v)


# device time: 55225 ns/iter; 1.0438x vs baseline; 1.0438x over previous
import math

import jax
import jax.numpy as jnp
from jax import lax
from jax.experimental import pallas as pl
from jax.experimental.pallas import tpu as pltpu

N_DEV = 16

_RING = [0, 4, 8, 12, 13, 9, 5, 1, 2, 6, 10, 14, 15, 11, 7, 3]
_INV = [0] * N_DEV
for _pos, _dev in enumerate(_RING):
    _INV[_dev] = _pos
_NEXT = [_RING[(_INV[d] + 1) % N_DEV] for d in range(N_DEV)]
_PREV = [_RING[(_INV[d] - 1) % N_DEV] for d in range(N_DEV)]

N_RIGHT = 8
N_LEFT = 7


def _select(idx, table):
    out = jnp.int32(table[0])
    for p in range(1, N_DEV):
        out = jnp.where(idx == p, jnp.int32(table[p]), out)
    return out


def kernel(q, k, v):
    s, d = q.shape
    scale = 1.0 / math.sqrt(d)

    def body(q_ref, k_ref, v_ref, out_ref, kbuf, vbuf, send_sems, recv_sems):
        my = lax.axis_index("i")
        right = _select(my, _NEXT)
        left = _select(my, _PREV)

        barrier_sem = pltpu.get_barrier_semaphore()
        for nbr in (left, right):
            pl.semaphore_signal(
                barrier_sem, inc=1,
                device_id=(nbr,), device_id_type=pl.DeviceIdType.MESH,
            )
        pl.semaphore_wait(barrier_sem, 2)

        qb = q_ref[...].astype(jnp.bfloat16)
        kbuf[0, :, :] = k_ref[...].astype(jnp.bfloat16)
        vbuf[0, :, :] = v_ref[...].astype(jnp.bfloat16)

        m = jnp.full((s, 1), -jnp.inf, dtype=jnp.float32)
        l = jnp.zeros((s, 1), dtype=jnp.float32)
        acc = jnp.zeros((s, d), dtype=jnp.float32)

        def phase_k(slot, m, l):
            kh = kbuf[slot, :, :]
            scores = lax.dot_general(
                qb, kh, (((1,), (1,)), ((), ())),
                preferred_element_type=jnp.float32,
            ) * scale
            m_new = jnp.maximum(m, jnp.max(scores, axis=1, keepdims=True))
            p = jnp.exp(scores - m_new)
            corr = jnp.exp(m - m_new)
            l = l * corr + jnp.sum(p, axis=1, keepdims=True)
            return m_new, l, p.astype(jnp.bfloat16), corr

        def phase_v(slot, acc, p, corr):
            vh = vbuf[slot, :, :]
            return acc * corr + lax.dot_general(
                p, vh, (((1,), (0,)), ((), ())),
                preferred_element_type=jnp.float32,
            )

        def rdma(buf, src_slot, dst_slot, sem_idx, dev):
            return pltpu.make_async_remote_copy(
                src_ref=buf.at[src_slot],
                dst_ref=buf.at[dst_slot],
                send_sem=send_sems.at[sem_idx],
                recv_sem=recv_sems.at[sem_idx],
                device_id=(dev,),
                device_id_type=pl.DeviceIdType.MESH,
            )

        started = []
        prev = {}

        def fwd(buf, src_slot, dst_slot, sem_idx, dev, key):
            r = rdma(buf, src_slot, dst_slot, sem_idx, dev)
            r.start()
            started.append(r)
            prev[key] = r

        for t in range(N_RIGHT + 1):
            if t > 0:
                prev[("r", "k")].wait_recv()
            if t < N_RIGHT:
                fwd(kbuf, t, t + 1, 4 * t + 0, right, ("r", "k"))
            if 0 < t <= N_LEFT:
                prev[("l", "k")].wait_recv()
            if t < N_LEFT:
                fwd(kbuf, 0 if t == 0 else 8 + t, 9 + t, 4 * t + 2, left,
                    ("l", "k"))

            slot_r = 0 if t == 0 else t
            m, l, p_r, corr_r = phase_k(slot_r, m, l)

            if t > 0:
                prev[("r", "v")].wait_recv()
            if t < N_RIGHT:
                fwd(vbuf, t, t + 1, 4 * t + 1, right, ("r", "v"))
            acc = phase_v(slot_r, acc, p_r, corr_r)

            if 0 < t <= N_LEFT:
                m, l, p_l, corr_l = phase_k(8 + t, m, l)
                prev[("l", "v")].wait_recv()
                if t < N_LEFT:
                    fwd(vbuf, 8 + t, 9 + t, 4 * t + 3, left, ("l", "v"))
                acc = phase_v(8 + t, acc, p_l, corr_l)
            elif t == 0:
                fwd(vbuf, 0, 9, 3, left, ("l", "v"))

        for r in started:
            r.wait_send()

        out_ref[...] = acc / l

    return pl.pallas_call(
        body,
        out_shape=jax.ShapeDtypeStruct((s, d), jnp.float32),
        in_specs=[
            pl.BlockSpec(memory_space=pltpu.VMEM),
            pl.BlockSpec(memory_space=pltpu.VMEM),
            pl.BlockSpec(memory_space=pltpu.VMEM),
        ],
        out_specs=pl.BlockSpec(memory_space=pltpu.VMEM),
        scratch_shapes=[
            pltpu.VMEM((N_DEV, s, d), jnp.bfloat16),
            pltpu.VMEM((N_DEV, s, d), jnp.bfloat16),
            pltpu.SemaphoreType.DMA((4 * N_RIGHT,)),
            pltpu.SemaphoreType.DMA((4 * N_RIGHT,)),
        ],
        compiler_params=pltpu.CompilerParams(collective_id=0),
    )(q, k, v)


# device time: 55126 ns/iter; 1.0457x vs baseline; 1.0018x over previous
import math

import jax
import jax.numpy as jnp
from jax import lax
from jax.experimental import pallas as pl
from jax.experimental.pallas import tpu as pltpu

N_DEV = 16

_RING = [0, 4, 8, 12, 13, 9, 5, 1, 2, 6, 10, 14, 15, 11, 7, 3]
_INV = [0] * N_DEV
for _pos, _dev in enumerate(_RING):
    _INV[_dev] = _pos
_NEXT = [_RING[(_INV[d] + 1) % N_DEV] for d in range(N_DEV)]
_PREV = [_RING[(_INV[d] - 1) % N_DEV] for d in range(N_DEV)]

N_RIGHT = 8
N_LEFT = 7


def _select(idx, table):
    out = jnp.int32(table[0])
    for p in range(1, N_DEV):
        out = jnp.where(idx == p, jnp.int32(table[p]), out)
    return out


def kernel(q, k, v):
    s, d = q.shape
    scale = 1.0 / math.sqrt(d)

    def body(q_ref, k_ref, v_ref, out_ref, kbuf, vbuf, send_sems, recv_sems):
        my = lax.axis_index("i")
        right = _select(my, _NEXT)
        left = _select(my, _PREV)

        barrier_sem = pltpu.get_barrier_semaphore()
        for nbr in (left, right):
            pl.semaphore_signal(
                barrier_sem, inc=1,
                device_id=(nbr,), device_id_type=pl.DeviceIdType.MESH,
            )
        pl.semaphore_wait(barrier_sem, 2)

        kbuf[0, :, :] = k_ref[...].astype(jnp.bfloat16)
        qb = None

        m = jnp.full((s, 1), -jnp.inf, dtype=jnp.float32)
        l = jnp.zeros((s, 1), dtype=jnp.float32)
        acc = jnp.zeros((s, d), dtype=jnp.float32)

        def phase_k(slot, m, l):
            kh = kbuf[slot, :, :]
            scores = lax.dot_general(
                qb, kh, (((1,), (1,)), ((), ())),
                preferred_element_type=jnp.float32,
            ) * scale
            m_new = jnp.maximum(m, jnp.max(scores, axis=1, keepdims=True))
            p = jnp.exp(scores - m_new)
            corr = jnp.exp(m - m_new)
            l = l * corr + jnp.sum(p, axis=1, keepdims=True)
            return m_new, l, p.astype(jnp.bfloat16), corr

        def phase_v(slot, acc, p, corr):
            vh = vbuf[slot, :, :]
            return acc * corr + lax.dot_general(
                p, vh, (((1,), (0,)), ((), ())),
                preferred_element_type=jnp.float32,
            )

        def rdma(buf, src_slot, dst_slot, sem_idx, dev):
            return pltpu.make_async_remote_copy(
                src_ref=buf.at[src_slot],
                dst_ref=buf.at[dst_slot],
                send_sem=send_sems.at[sem_idx],
                recv_sem=recv_sems.at[sem_idx],
                device_id=(dev,),
                device_id_type=pl.DeviceIdType.MESH,
            )

        started = []
        prev = {}

        def fwd(buf, src_slot, dst_slot, sem_idx, dev, key):
            r = rdma(buf, src_slot, dst_slot, sem_idx, dev)
            r.start()
            started.append(r)
            prev[key] = r

        for t in range(N_RIGHT + 1):
            if t > 0:
                prev[("r", "k")].wait_recv()
            if t < N_RIGHT:
                fwd(kbuf, t, t + 1, 4 * t + 0, right, ("r", "k"))
            if 0 < t <= N_LEFT:
                prev[("l", "k")].wait_recv()
            if t < N_LEFT:
                fwd(kbuf, 0 if t == 0 else 8 + t, 9 + t, 4 * t + 2, left,
                    ("l", "k"))

            if t == 0:
                qb = q_ref[...].astype(jnp.bfloat16)
                vbuf[0, :, :] = v_ref[...].astype(jnp.bfloat16)
                fwd(vbuf, 0, 1, 1, right, ("r", "v"))
                fwd(vbuf, 0, 9, 3, left, ("l", "v"))

            slot_r = 0 if t == 0 else t
            m, l, p_r, corr_r = phase_k(slot_r, m, l)

            if t > 0:
                prev[("r", "v")].wait_recv()
                if t < N_RIGHT:
                    fwd(vbuf, t, t + 1, 4 * t + 1, right, ("r", "v"))
            acc = phase_v(slot_r, acc, p_r, corr_r)

            if 0 < t <= N_LEFT:
                m, l, p_l, corr_l = phase_k(8 + t, m, l)
                prev[("l", "v")].wait_recv()
                if t < N_LEFT:
                    fwd(vbuf, 8 + t, 9 + t, 4 * t + 3, left, ("l", "v"))
                acc = phase_v(8 + t, acc, p_l, corr_l)

        for r in started:
            r.wait_send()

        out_ref[...] = acc / l

    return pl.pallas_call(
        body,
        out_shape=jax.ShapeDtypeStruct((s, d), jnp.float32),
        in_specs=[
            pl.BlockSpec(memory_space=pltpu.VMEM),
            pl.BlockSpec(memory_space=pltpu.VMEM),
            pl.BlockSpec(memory_space=pltpu.VMEM),
        ],
        out_specs=pl.BlockSpec(memory_space=pltpu.VMEM),
        scratch_shapes=[
            pltpu.VMEM((N_DEV, s, d), jnp.bfloat16),
            pltpu.VMEM((N_DEV, s, d), jnp.bfloat16),
            pltpu.SemaphoreType.DMA((4 * N_RIGHT,)),
            pltpu.SemaphoreType.DMA((4 * N_RIGHT,)),
        ],
        compiler_params=pltpu.CompilerParams(collective_id=0),
    )(q, k, v)
